# baseline (speedup 1.0000x reference)
import jax
import jax.numpy as jnp
from jax.experimental import pallas as pl
from jax.experimental.pallas import tpu as pltpu

_N_CHUNKS = 16


def _folded_mlp_kernel(x_hbm, w0_ref, b0_ref, w1_ref, b1_ref, wout_ref,
                       bout_ref, out_hbm, xbuf, obuf, in_sems, out_sems):
    batch = x_hbm.shape[0]
    rows = batch // _N_CHUNKS

    def in_copy(i):
        return pltpu.make_async_copy(
            x_hbm.at[pl.ds(i * rows, rows), :], xbuf.at[i], in_sems.at[i])

    def out_copy(i):
        return pltpu.make_async_copy(
            obuf.at[i], out_hbm.at[pl.ds(i * rows, rows), :], out_sems.at[i])

    for i in range(_N_CHUNKS):
        in_copy(i).start()

    w01 = jnp.dot(w0_ref[...], w1_ref[...],
                  preferred_element_type=jnp.float32)
    w_eff = jnp.dot(w01, wout_ref[...], preferred_element_type=jnp.float32)
    b01 = jnp.dot(b0_ref[...], w1_ref[...],
                  preferred_element_type=jnp.float32) + b1_ref[...]
    b_eff = jnp.dot(b01, wout_ref[...],
                    preferred_element_type=jnp.float32) + bout_ref[...]

    for i in range(_N_CHUNKS):
        in_copy(i).wait()
        obuf[i] = jnp.dot(xbuf[i], w_eff,
                          preferred_element_type=jnp.float32) + b_eff
        out_copy(i).start()

    for i in range(_N_CHUNKS):
        out_copy(i).wait()


@jax.jit
def _run(x, W0, b0, W1, b1, W_out, b_out):
    batch, in_dim = x.shape
    hidden = W0.shape[1]
    out_dim = W_out.shape[1]
    rows = batch // _N_CHUNKS

    b0r = b0.reshape(1, hidden)
    b1r = b1.reshape(1, hidden)
    boutr = b_out.reshape(1, out_dim)

    full = lambda shape: pl.BlockSpec(shape, lambda: (0, 0))
    return pl.pallas_call(
        _folded_mlp_kernel,
        in_specs=[
            pl.BlockSpec(memory_space=pl.ANY),
            full((in_dim, hidden)),
            full((1, hidden)),
            full((hidden, hidden)),
            full((1, hidden)),
            full((hidden, out_dim)),
            full((1, out_dim)),
        ],
        out_specs=pl.BlockSpec(memory_space=pl.ANY),
        out_shape=jax.ShapeDtypeStruct((batch, out_dim), jnp.float32),
        scratch_shapes=[
            pltpu.VMEM((_N_CHUNKS, rows, in_dim), jnp.float32),
            pltpu.VMEM((_N_CHUNKS, rows, out_dim), jnp.float32),
            pltpu.SemaphoreType.DMA((_N_CHUNKS,)),
            pltpu.SemaphoreType.DMA((_N_CHUNKS,)),
        ],
    )(x, W0, b0r, W1, b1r, W_out, boutr)


def kernel(x, W0, b0, W1, b1, W_out, b_out):
    return _run(x, W0, b0, W1, b1, W_out, b_out)


# speedup vs baseline: 1.0501x; 1.0501x over previous
import jax
import jax.numpy as jnp
from jax.experimental import pallas as pl
from jax.experimental.pallas import tpu as pltpu

_N_CHUNKS = 4


def _folded_mlp_kernel(x_hbm, w0_ref, b0_ref, w1_ref, b1_ref, wout_ref,
                       bout_ref, out_hbm, xbuf, obuf, in_sems, out_sems):
    batch = x_hbm.shape[0]
    rows = batch // _N_CHUNKS

    def in_copy(i):
        return pltpu.make_async_copy(
            x_hbm.at[pl.ds(i * rows, rows), :], xbuf.at[i], in_sems.at[i])

    def out_copy(i):
        return pltpu.make_async_copy(
            obuf.at[i], out_hbm.at[pl.ds(i * rows, rows), :], out_sems.at[i])

    for i in range(_N_CHUNKS):
        in_copy(i).start()

    w01 = jnp.dot(w0_ref[...], w1_ref[...],
                  preferred_element_type=jnp.float32)
    w_eff = jnp.dot(w01, wout_ref[...], preferred_element_type=jnp.float32)
    b01 = jnp.dot(b0_ref[...], w1_ref[...],
                  preferred_element_type=jnp.float32) + b1_ref[...]
    b_eff = jnp.dot(b01, wout_ref[...],
                    preferred_element_type=jnp.float32) + bout_ref[...]

    for i in range(_N_CHUNKS):
        in_copy(i).wait()
        obuf[i] = jnp.dot(xbuf[i], w_eff,
                          preferred_element_type=jnp.float32) + b_eff
        out_copy(i).start()

    for i in range(_N_CHUNKS):
        out_copy(i).wait()


@jax.jit
def _run(x, W0, b0, W1, b1, W_out, b_out):
    batch, in_dim = x.shape
    hidden = W0.shape[1]
    out_dim = W_out.shape[1]
    rows = batch // _N_CHUNKS

    b0r = b0.reshape(1, hidden)
    b1r = b1.reshape(1, hidden)
    boutr = b_out.reshape(1, out_dim)

    full = lambda shape: pl.BlockSpec(shape, lambda: (0, 0))
    return pl.pallas_call(
        _folded_mlp_kernel,
        in_specs=[
            pl.BlockSpec(memory_space=pl.ANY),
            full((in_dim, hidden)),
            full((1, hidden)),
            full((hidden, hidden)),
            full((1, hidden)),
            full((hidden, out_dim)),
            full((1, out_dim)),
        ],
        out_specs=pl.BlockSpec(memory_space=pl.ANY),
        out_shape=jax.ShapeDtypeStruct((batch, out_dim), jnp.float32),
        scratch_shapes=[
            pltpu.VMEM((_N_CHUNKS, rows, in_dim), jnp.float32),
            pltpu.VMEM((_N_CHUNKS, rows, out_dim), jnp.float32),
            pltpu.SemaphoreType.DMA((_N_CHUNKS,)),
            pltpu.SemaphoreType.DMA((_N_CHUNKS,)),
        ],
    )(x, W0, b0r, W1, b1r, W_out, boutr)


def kernel(x, W0, b0, W1, b1, W_out, b_out):
    return _run(x, W0, b0, W1, b1, W_out, b_out)


# speedup vs baseline: 1.0568x; 1.0063x over previous
import jax
import jax.numpy as jnp
from jax.experimental import pallas as pl
from jax.experimental.pallas import tpu as pltpu

_N_CHUNKS = 8


def _folded_mlp_kernel(x_hbm, w0_ref, b0_ref, w1_ref, b1_ref, wout_ref,
                       bout_ref, out_hbm, xbuf, obuf, in_sems, out_sems):
    batch = x_hbm.shape[0]
    rows = batch // _N_CHUNKS

    def in_copy(i):
        return pltpu.make_async_copy(
            x_hbm.at[pl.ds(i * rows, rows), :], xbuf.at[i], in_sems.at[i])

    def out_copy(i):
        return pltpu.make_async_copy(
            obuf.at[i], out_hbm.at[pl.ds(i * rows, rows), :], out_sems.at[i])

    for i in range(_N_CHUNKS):
        in_copy(i).start()

    w01 = jnp.dot(w0_ref[...], w1_ref[...],
                  preferred_element_type=jnp.float32)
    w_eff = jnp.dot(w01, wout_ref[...], preferred_element_type=jnp.float32)
    b01 = jnp.dot(b0_ref[...], w1_ref[...],
                  preferred_element_type=jnp.float32) + b1_ref[...]
    b_eff = jnp.dot(b01, wout_ref[...],
                    preferred_element_type=jnp.float32) + bout_ref[...]

    for i in range(_N_CHUNKS):
        in_copy(i).wait()
        obuf[i] = jnp.dot(xbuf[i], w_eff,
                          preferred_element_type=jnp.float32) + b_eff
        out_copy(i).start()

    for i in range(_N_CHUNKS):
        out_copy(i).wait()


@jax.jit
def _run(x, W0, b0, W1, b1, W_out, b_out):
    batch, in_dim = x.shape
    hidden = W0.shape[1]
    out_dim = W_out.shape[1]
    rows = batch // _N_CHUNKS

    b0r = b0.reshape(1, hidden)
    b1r = b1.reshape(1, hidden)
    boutr = b_out.reshape(1, out_dim)

    full = lambda shape: pl.BlockSpec(shape, lambda: (0, 0))
    return pl.pallas_call(
        _folded_mlp_kernel,
        in_specs=[
            pl.BlockSpec(memory_space=pl.ANY),
            full((in_dim, hidden)),
            full((1, hidden)),
            full((hidden, hidden)),
            full((1, hidden)),
            full((hidden, out_dim)),
            full((1, out_dim)),
        ],
        out_specs=pl.BlockSpec(memory_space=pl.ANY),
        out_shape=jax.ShapeDtypeStruct((batch, out_dim), jnp.float32),
        scratch_shapes=[
            pltpu.VMEM((_N_CHUNKS, rows, in_dim), jnp.float32),
            pltpu.VMEM((_N_CHUNKS, rows, out_dim), jnp.float32),
            pltpu.SemaphoreType.DMA((_N_CHUNKS,)),
            pltpu.SemaphoreType.DMA((_N_CHUNKS,)),
        ],
        compiler_params=pltpu.CompilerParams(
            skip_device_barrier=True,
            disable_bounds_checks=True,
        ),
    )(x, W0, b0r, W1, b1r, W_out, boutr)


def kernel(x, W0, b0, W1, b1, W_out, b_out):
    return _run(x, W0, b0, W1, b1, W_out, b_out)
